# Initial kernel scaffold; baseline (speedup 1.0000x reference)
import jax
import jax.numpy as jnp
from jax import lax
from jax.experimental import pallas as pl
from jax.experimental.pallas import tpu as pltpu
from jax.experimental.pallas import tpu_sc as plsc

NE = 320000
D = 128
NSEG = 10000
NC = 2
NS = 16
L = 16

DH = D // NC
SEG_PAD = 10240
RPT = SEG_PAD // NS

IDX_COLS = 128
IDX_ROWS = NE // IDX_COLS
BASE_ROWS = IDX_ROWS // NS
EXTRA = IDX_ROWS - BASE_ROWS * NS
RPI = 4
STEPS = BASE_ROWS // RPI


def _sc_body(msg_hbm, idx_hbm, out_hbm, acc, cstage, mbuf, ibuf, counts,
             sums, cpart, recip, zbuf):
    c = lax.axis_index("c")
    s = lax.axis_index("s")
    col0 = c * DH
    seg0 = s * RPT

    zero16 = jnp.zeros((L,), jnp.float32)
    ones16 = jnp.full((L,), 1.0, jnp.float32)

    def _zc(i, carry):
        counts[pl.ds(i * L, L)] = zero16
        return carry
    lax.fori_loop(0, SEG_PAD // L, _zc, None)
    for i in range(L):
        for j in range(DH // L):
            zbuf[i, pl.ds(j * L, L)] = zero16
    def _za(q, carry):
        pltpu.sync_copy(zbuf, acc.at[pl.ds(seg0 + q * L, L)])
        return carry
    lax.fori_loop(0, RPT // L, _za, None)
    plsc.subcore_barrier()

    def _count(j):
        for q in range(IDX_COLS // L):
            iv = ibuf[j, pl.ds(q * L, L)]
            plsc.addupdate_scatter(counts, [iv], ones16)

    def _step(k, carry):
        row = s * BASE_ROWS + k * RPI
        e0 = row * IDX_COLS
        pltpu.sync_copy(idx_hbm.at[pl.ds(row, RPI)], ibuf)
        pltpu.sync_copy(msg_hbm.at[pl.ds(e0, RPI * IDX_COLS), pl.ds(col0, DH)],
                        mbuf)
        for j in range(RPI):
            pltpu.sync_copy(mbuf.at[pl.ds(j * IDX_COLS, IDX_COLS)],
                            acc.at[ibuf.at[j]], add=True)
            _count(j)
        return carry
    lax.fori_loop(0, STEPS, _step, None)

    @pl.when(s < EXTRA)
    def _extra():
        row = NS * BASE_ROWS + s
        e0 = row * IDX_COLS
        pltpu.sync_copy(idx_hbm.at[pl.ds(row, 1)], ibuf.at[pl.ds(0, 1)])
        pltpu.sync_copy(msg_hbm.at[pl.ds(e0, IDX_COLS), pl.ds(col0, DH)],
                        mbuf.at[pl.ds(0, IDX_COLS)])
        pltpu.sync_copy(mbuf.at[pl.ds(0, IDX_COLS)], acc.at[ibuf.at[0]],
                        add=True)
        _count(0)

    pltpu.sync_copy(counts, cstage.at[s])
    plsc.subcore_barrier()

    pltpu.sync_copy(cstage.at[:, pl.ds(seg0, RPT)], cpart)
    def _merge(r, carry):
        tot = zero16
        for t_ in range(NS):
            tot = tot + cpart[t_, pl.ds(r * L, L)]
        recip[pl.ds(r * L, L)] = ones16 / jnp.maximum(tot, ones16)
        return carry
    lax.fori_loop(0, RPT // L, _merge, None)

    pltpu.sync_copy(acc.at[pl.ds(seg0, RPT)], sums)
    def _div(r, carry):
        rv = plsc.load_gather(recip, [jnp.full((L,), r, jnp.int32)])
        for j in range(DH // L):
            sums[r, pl.ds(j * L, L)] = sums[r, pl.ds(j * L, L)] * rv
        return carry
    lax.fori_loop(0, RPT, _div, None)
    pltpu.sync_copy(sums, out_hbm.at[pl.ds(seg0, RPT), pl.ds(col0, DH)])


@jax.jit
def kernel(msg, index, t):
    del t
    idx2d = index.astype(jnp.int32).reshape(IDX_ROWS, IDX_COLS)
    mesh = plsc.VectorSubcoreMesh(core_axis_name="c", subcore_axis_name="s",
                                  num_cores=NC, num_subcores=NS)
    out = pl.kernel(
        _sc_body,
        out_type=jax.ShapeDtypeStruct((SEG_PAD, D), jnp.float32),
        mesh=mesh,
        scratch_types=[
            pltpu.VMEM_SHARED((SEG_PAD, DH), jnp.float32),
            pltpu.VMEM_SHARED((NS, SEG_PAD), jnp.float32),
            pltpu.VMEM((RPI * IDX_COLS, DH), jnp.float32),
            pltpu.VMEM((RPI, IDX_COLS), jnp.int32),
            pltpu.VMEM((SEG_PAD,), jnp.float32),
            pltpu.VMEM((RPT, DH), jnp.float32),
            pltpu.VMEM((NS, RPT), jnp.float32),
            pltpu.VMEM((RPT,), jnp.float32),
            pltpu.VMEM((L, DH), jnp.float32),
        ],
    )(msg, idx2d)
    return out[:NSEG]


# speedup vs baseline: 6.1304x; 6.1304x over previous
import jax
import jax.numpy as jnp
from jax import lax
from jax.experimental import pallas as pl
from jax.experimental.pallas import tpu as pltpu
from jax.experimental.pallas import tpu_sc as plsc

NE = 320000
D = 128
NSEG = 10000
NC = 2
NS = 16
L = 16

DH = D // NC
SEG_PAD = 10240
RPT = SEG_PAD // NS

IDX_COLS = 128
IDX_ROWS = NE // IDX_COLS
BASE_ROWS = IDX_ROWS // NS
EXTRA = IDX_ROWS - BASE_ROWS * NS
RPI = 4
STEPS = BASE_ROWS // RPI


def _sc_body(msg_hbm, idx_hbm, out0_hbm, out1_hbm, acc, cacc, mbuf, ibuf,
             ones_buf, ccols, zbuf):
    c = lax.axis_index("c")
    s = lax.axis_index("s")
    col0 = c * DH
    seg0 = s * RPT

    zero16 = jnp.zeros((L,), jnp.float32)
    ones16 = jnp.full((L,), 1.0, jnp.float32)

    for i in range(64):
        for j in range(DH // L):
            zbuf[i, pl.ds(j * L, L)] = zero16
    def _zo(i, carry):
        ones_buf[i, pl.ds(0, L)] = ones16
        return carry
    lax.fori_loop(0, IDX_COLS, _zo, None)
    def _za(q, carry):
        pltpu.sync_copy(zbuf, acc.at[pl.ds(seg0 + q * 64, 64)])
        pltpu.sync_copy(zbuf.at[:, pl.ds(0, L)],
                        cacc.at[pl.ds(seg0 + q * 64, 64)])
        return carry
    lax.fori_loop(0, RPT // 64, _za, None)
    plsc.subcore_barrier()

    def _step(k, carry):
        row = s * BASE_ROWS + k * RPI
        e0 = row * IDX_COLS
        pltpu.sync_copy(idx_hbm.at[pl.ds(row, RPI)], ibuf)
        pltpu.sync_copy(msg_hbm.at[pl.ds(e0, RPI * IDX_COLS), pl.ds(col0, DH)],
                        mbuf.at[pl.ds(0, RPI * IDX_COLS)])
        for j in range(RPI):
            pltpu.sync_copy(mbuf.at[pl.ds(j * IDX_COLS, IDX_COLS)],
                            acc.at[ibuf.at[j]], add=True)
            pltpu.sync_copy(ones_buf, cacc.at[ibuf.at[j]], add=True)
        return carry
    lax.fori_loop(0, STEPS, _step, None)

    @pl.when(s < EXTRA)
    def _extra():
        row = NS * BASE_ROWS + s
        e0 = row * IDX_COLS
        pltpu.sync_copy(idx_hbm.at[pl.ds(row, 1)], ibuf.at[pl.ds(0, 1)])
        pltpu.sync_copy(msg_hbm.at[pl.ds(e0, IDX_COLS), pl.ds(col0, DH)],
                        mbuf.at[pl.ds(0, IDX_COLS)])
        pltpu.sync_copy(mbuf.at[pl.ds(0, IDX_COLS)], acc.at[ibuf.at[0]],
                        add=True)
        pltpu.sync_copy(ones_buf, cacc.at[ibuf.at[0]], add=True)

    plsc.subcore_barrier()

    pltpu.sync_copy(acc.at[pl.ds(seg0, RPT)], mbuf)
    pltpu.sync_copy(cacc.at[pl.ds(seg0, RPT)], ccols)
    def _div(r, carry):
        cnt = ccols[r, pl.ds(0, L)]
        rv = ones16 / jnp.maximum(cnt, ones16)
        for j in range(DH // L):
            mbuf[r, pl.ds(j * L, L)] = mbuf[r, pl.ds(j * L, L)] * rv
        return carry
    lax.fori_loop(0, RPT, _div, None)

    @pl.when(c == 0)
    def _w0():
        pltpu.sync_copy(mbuf, out0_hbm.at[pl.ds(seg0, RPT)])

    @pl.when(c == 1)
    def _w1():
        pltpu.sync_copy(mbuf, out1_hbm.at[pl.ds(seg0, RPT)])


@jax.jit
def kernel(msg, index, t):
    del t
    idx2d = index.astype(jnp.int32).reshape(IDX_ROWS, IDX_COLS)
    mesh = plsc.VectorSubcoreMesh(core_axis_name="c", subcore_axis_name="s",
                                  num_cores=NC, num_subcores=NS)
    out = pl.kernel(
        _sc_body,
        out_type=(jax.ShapeDtypeStruct((SEG_PAD, DH), jnp.float32),
                  jax.ShapeDtypeStruct((SEG_PAD, DH), jnp.float32)),
        mesh=mesh,
        compiler_params=pltpu.CompilerParams(use_tc_tiling_on_sc=False,
                                             needs_layout_passes=False),
        scratch_types=[
            pltpu.VMEM_SHARED((SEG_PAD, DH), jnp.float32),
            pltpu.VMEM_SHARED((SEG_PAD, L), jnp.float32),
            pltpu.VMEM((RPT, DH), jnp.float32),
            pltpu.VMEM((RPI, IDX_COLS), jnp.int32),
            pltpu.VMEM((IDX_COLS, L), jnp.float32),
            pltpu.VMEM((RPT, L), jnp.float32),
            pltpu.VMEM((64, DH), jnp.float32),
        ],
    )(msg, idx2d)
    return jnp.concatenate([out[0][:NSEG], out[1][:NSEG]], axis=1)


# speedup vs baseline: 9.2319x; 1.5059x over previous
import jax
import jax.numpy as jnp
from jax import lax
from jax.experimental import pallas as pl
from jax.experimental.pallas import tpu as pltpu
from jax.experimental.pallas import tpu_sc as plsc

NE = 320000
D = 128
NSEG = 10000
NC = 2
NS = 16
L = 16

DH = D // NC
SEG_PAD = 10240
RPT = SEG_PAD // NS

IDX_COLS = 128
IDX_ROWS = NE // IDX_COLS
BASE_ROWS = IDX_ROWS // NS
EXTRA = IDX_ROWS - BASE_ROWS * NS
RPI = 3
STEPS = BASE_ROWS // RPI
CHUNK = RPI * IDX_COLS


def _sc_body(msg_hbm, idx_hbm, out0_hbm, out1_hbm, acc, cacc,
             buf0, buf1, ib0, ib1, ones_buf, ccols, zbuf,
             si0, sm0, si1, sm1):
    c = lax.axis_index("c")
    s = lax.axis_index("s")
    col0 = c * DH
    seg0 = s * RPT
    row_base = s * BASE_ROWS

    zero16 = jnp.zeros((L,), jnp.float32)
    ones16 = jnp.full((L,), 1.0, jnp.float32)

    for i in range(L):
        for j in range(DH // L):
            zbuf[i, pl.ds(j * L, L)] = zero16
    def _zo(i, carry):
        ones_buf[i, pl.ds(0, L)] = ones16
        return carry
    lax.fori_loop(0, IDX_COLS, _zo, None)
    def _za(q, carry):
        pltpu.sync_copy(zbuf, acc.at[pl.ds(seg0 + q * L, L)])
        pltpu.sync_copy(zbuf.at[:, pl.ds(0, L)],
                        cacc.at[pl.ds(seg0 + q * L, L)])
        return carry
    lax.fori_loop(0, RPT // L, _za, None)
    plsc.subcore_barrier()

    def _start(row, ib, buf, si, sm):
        pltpu.async_copy(idx_hbm.at[pl.ds(row, RPI)], ib, si)
        pltpu.async_copy(
            msg_hbm.at[pl.ds(row * IDX_COLS, CHUNK), pl.ds(col0, DH)],
            buf, sm)

    def _wait(row, ib, buf, si, sm):
        pltpu.make_async_copy(idx_hbm.at[pl.ds(row, RPI)], ib, si).wait()
        pltpu.make_async_copy(
            msg_hbm.at[pl.ds(row * IDX_COLS, CHUNK), pl.ds(col0, DH)],
            buf, sm).wait()

    def _adds(ib, buf):
        for j in range(RPI):
            pltpu.sync_copy(buf.at[pl.ds(j * IDX_COLS, IDX_COLS)],
                            acc.at[ib.at[j]], add=True)
            pltpu.sync_copy(ones_buf, cacc.at[ib.at[j]], add=True)

    _start(row_base, ib0, buf0, si0, sm0)
    def _pair(p, carry):
        row_a = row_base + (2 * p) * RPI
        _start(row_a + RPI, ib1, buf1, si1, sm1)
        _wait(row_a, ib0, buf0, si0, sm0)
        _adds(ib0, buf0)

        @pl.when(p < STEPS // 2 - 1)
        def _():
            _start(row_a + 2 * RPI, ib0, buf0, si0, sm0)
        _wait(row_a + RPI, ib1, buf1, si1, sm1)
        _adds(ib1, buf1)
        return carry
    lax.fori_loop(0, STEPS // 2, _pair, None)

    @pl.when(s < EXTRA)
    def _extra():
        row = NS * BASE_ROWS + s
        e0 = row * IDX_COLS
        pltpu.sync_copy(idx_hbm.at[pl.ds(row, 1)], ib0.at[pl.ds(0, 1)])
        pltpu.sync_copy(msg_hbm.at[pl.ds(e0, IDX_COLS), pl.ds(col0, DH)],
                        buf0.at[pl.ds(0, IDX_COLS)])
        pltpu.sync_copy(buf0.at[pl.ds(0, IDX_COLS)], acc.at[ib0.at[0]],
                        add=True)
        pltpu.sync_copy(ones_buf, cacc.at[ib0.at[0]], add=True)

    plsc.subcore_barrier()

    pltpu.sync_copy(cacc.at[pl.ds(seg0, RPT)], ccols)
    for start, n in ((0, CHUNK), (CHUNK, RPT - CHUNK)):
        pltpu.sync_copy(acc.at[pl.ds(seg0 + start, n)], buf0.at[pl.ds(0, n)])
        def _div(r, carry, start=start):
            cnt = ccols[start + r, pl.ds(0, L)]
            rv = ones16 / jnp.maximum(cnt, ones16)
            for j in range(DH // L):
                buf0[r, pl.ds(j * L, L)] = buf0[r, pl.ds(j * L, L)] * rv
            return carry
        lax.fori_loop(0, n, _div, None)

        @pl.when(c == 0)
        def _w0(start=start, n=n):
            pltpu.sync_copy(buf0.at[pl.ds(0, n)],
                            out0_hbm.at[pl.ds(seg0 + start, n)])

        @pl.when(c == 1)
        def _w1(start=start, n=n):
            pltpu.sync_copy(buf0.at[pl.ds(0, n)],
                            out1_hbm.at[pl.ds(seg0 + start, n)])


@jax.jit
def kernel(msg, index, t):
    del t
    idx2d = index.astype(jnp.int32).reshape(IDX_ROWS, IDX_COLS)
    mesh = plsc.VectorSubcoreMesh(core_axis_name="c", subcore_axis_name="s",
                                  num_cores=NC, num_subcores=NS)
    out = pl.kernel(
        _sc_body,
        out_type=(jax.ShapeDtypeStruct((SEG_PAD, DH), jnp.float32),
                  jax.ShapeDtypeStruct((SEG_PAD, DH), jnp.float32)),
        mesh=mesh,
        compiler_params=pltpu.CompilerParams(use_tc_tiling_on_sc=False,
                                             needs_layout_passes=False),
        scratch_types=[
            pltpu.VMEM_SHARED((SEG_PAD, DH), jnp.float32),
            pltpu.VMEM_SHARED((SEG_PAD, L), jnp.float32),
            pltpu.VMEM((CHUNK, DH), jnp.float32),
            pltpu.VMEM((CHUNK, DH), jnp.float32),
            pltpu.VMEM((RPI, IDX_COLS), jnp.int32),
            pltpu.VMEM((RPI, IDX_COLS), jnp.int32),
            pltpu.VMEM((IDX_COLS, L), jnp.float32),
            pltpu.VMEM((RPT, L), jnp.float32),
            pltpu.VMEM((L, DH), jnp.float32),
            pltpu.SemaphoreType.DMA,
            pltpu.SemaphoreType.DMA,
            pltpu.SemaphoreType.DMA,
            pltpu.SemaphoreType.DMA,
        ],
    )(msg, idx2d)
    return jnp.concatenate([out[0][:NSEG], out[1][:NSEG]], axis=1)


# speedup vs baseline: 9.3713x; 1.0151x over previous
import jax
import jax.numpy as jnp
from jax import lax
from jax.experimental import pallas as pl
from jax.experimental.pallas import tpu as pltpu
from jax.experimental.pallas import tpu_sc as plsc

NE = 320000
D = 128
NSEG = 10000
NC = 2
NS = 16
L = 16

DH = D // NC
SEG_PAD = 10240
RPT = SEG_PAD // NS

IDX_COLS = 128
IDX_ROWS = NE // IDX_COLS
BASE_ROWS = IDX_ROWS // NS
EXTRA = IDX_ROWS - BASE_ROWS * NS
RPI = 3
STEPS = BASE_ROWS // RPI
CHUNK = RPI * IDX_COLS


def _sc_body(msg_hbm, idx_hbm, out0_hbm, out1_hbm, acc, cacc,
             buf0, buf1, ib0, ib1, ones_buf, ccols, zbuf,
             si0, sm0, si1, sm1, sa0, sa1):
    c = lax.axis_index("c")
    s = lax.axis_index("s")
    col0 = c * DH
    seg0 = s * RPT
    row_base = s * BASE_ROWS

    zero16 = jnp.zeros((L,), jnp.float32)
    ones16 = jnp.full((L,), 1.0, jnp.float32)

    for i in range(L):
        for j in range(DH // L):
            zbuf[i, pl.ds(j * L, L)] = zero16
    def _zo(i, carry):
        ones_buf[i, pl.ds(0, L)] = ones16
        return carry
    lax.fori_loop(0, IDX_COLS, _zo, None)
    def _za(q, carry):
        pltpu.sync_copy(zbuf, acc.at[pl.ds(seg0 + q * L, L)])
        pltpu.sync_copy(zbuf.at[:, pl.ds(0, L)],
                        cacc.at[pl.ds(seg0 + q * L, L)])
        return carry
    lax.fori_loop(0, RPT // L, _za, None)
    plsc.subcore_barrier()

    def _start(row, ib, buf, si, sm):
        pltpu.async_copy(idx_hbm.at[pl.ds(row, RPI)], ib, si)
        pltpu.async_copy(
            msg_hbm.at[pl.ds(row * IDX_COLS, CHUNK), pl.ds(col0, DH)],
            buf, sm)

    def _wait(row, ib, buf, si, sm):
        pltpu.make_async_copy(idx_hbm.at[pl.ds(row, RPI)], ib, si).wait()
        pltpu.make_async_copy(
            msg_hbm.at[pl.ds(row * IDX_COLS, CHUNK), pl.ds(col0, DH)],
            buf, sm).wait()

    def _adds(ib, buf, sa):
        descs = []
        for j in range(RPI):
            descs.append(pltpu.async_copy(
                buf.at[pl.ds(j * IDX_COLS, IDX_COLS)],
                acc.at[ib.at[j]], sa, add=True))
            descs.append(pltpu.async_copy(
                ones_buf, cacc.at[ib.at[j]], sa, add=True))
        return descs

    _start(row_base, ib0, buf0, si0, sm0)
    def _pair(p, carry):
        row_a = row_base + (2 * p) * RPI
        _start(row_a + RPI, ib1, buf1, si1, sm1)
        _wait(row_a, ib0, buf0, si0, sm0)
        for d in _adds(ib0, buf0, sa0):
            d.wait()

        @pl.when(p < STEPS // 2 - 1)
        def _():
            _start(row_a + 2 * RPI, ib0, buf0, si0, sm0)
        _wait(row_a + RPI, ib1, buf1, si1, sm1)
        for d in _adds(ib1, buf1, sa1):
            d.wait()
        return carry
    lax.fori_loop(0, STEPS // 2, _pair, None)

    @pl.when(s < EXTRA)
    def _extra():
        row = NS * BASE_ROWS + s
        e0 = row * IDX_COLS
        pltpu.sync_copy(idx_hbm.at[pl.ds(row, 1)], ib0.at[pl.ds(0, 1)])
        pltpu.sync_copy(msg_hbm.at[pl.ds(e0, IDX_COLS), pl.ds(col0, DH)],
                        buf0.at[pl.ds(0, IDX_COLS)])
        pltpu.sync_copy(buf0.at[pl.ds(0, IDX_COLS)], acc.at[ib0.at[0]],
                        add=True)
        pltpu.sync_copy(ones_buf, cacc.at[ib0.at[0]], add=True)

    plsc.subcore_barrier()

    pltpu.sync_copy(cacc.at[pl.ds(seg0, RPT)], ccols)
    for start, n in ((0, CHUNK), (CHUNK, RPT - CHUNK)):
        pltpu.sync_copy(acc.at[pl.ds(seg0 + start, n)], buf0.at[pl.ds(0, n)])
        def _div(r, carry, start=start):
            cnt = ccols[start + r, pl.ds(0, L)]
            rv = ones16 / jnp.maximum(cnt, ones16)
            for j in range(DH // L):
                buf0[r, pl.ds(j * L, L)] = buf0[r, pl.ds(j * L, L)] * rv
            return carry
        lax.fori_loop(0, n, _div, None)

        @pl.when(c == 0)
        def _w0(start=start, n=n):
            pltpu.sync_copy(buf0.at[pl.ds(0, n)],
                            out0_hbm.at[pl.ds(seg0 + start, n)])

        @pl.when(c == 1)
        def _w1(start=start, n=n):
            pltpu.sync_copy(buf0.at[pl.ds(0, n)],
                            out1_hbm.at[pl.ds(seg0 + start, n)])


@jax.jit
def kernel(msg, index, t):
    del t
    idx2d = index.astype(jnp.int32).reshape(IDX_ROWS, IDX_COLS)
    mesh = plsc.VectorSubcoreMesh(core_axis_name="c", subcore_axis_name="s",
                                  num_cores=NC, num_subcores=NS)
    out = pl.kernel(
        _sc_body,
        out_type=(jax.ShapeDtypeStruct((SEG_PAD, DH), jnp.float32),
                  jax.ShapeDtypeStruct((SEG_PAD, DH), jnp.float32)),
        mesh=mesh,
        compiler_params=pltpu.CompilerParams(use_tc_tiling_on_sc=False,
                                             needs_layout_passes=False),
        scratch_types=[
            pltpu.VMEM_SHARED((SEG_PAD, DH), jnp.float32),
            pltpu.VMEM_SHARED((SEG_PAD, L), jnp.float32),
            pltpu.VMEM((CHUNK, DH), jnp.float32),
            pltpu.VMEM((CHUNK, DH), jnp.float32),
            pltpu.VMEM((RPI, IDX_COLS), jnp.int32),
            pltpu.VMEM((RPI, IDX_COLS), jnp.int32),
            pltpu.VMEM((IDX_COLS, L), jnp.float32),
            pltpu.VMEM((RPT, L), jnp.float32),
            pltpu.VMEM((L, DH), jnp.float32),
            pltpu.SemaphoreType.DMA,
            pltpu.SemaphoreType.DMA,
            pltpu.SemaphoreType.DMA,
            pltpu.SemaphoreType.DMA,
            pltpu.SemaphoreType.DMA,
            pltpu.SemaphoreType.DMA,
        ],
    )(msg, idx2d)
    return jnp.concatenate([out[0][:NSEG], out[1][:NSEG]], axis=1)


# speedup vs baseline: 10.0546x; 1.0729x over previous
import jax
import jax.numpy as jnp
from jax import lax
from jax.experimental import pallas as pl
from jax.experimental.pallas import tpu as pltpu
from jax.experimental.pallas import tpu_sc as plsc

NE = 320000
D = 128
NSEG = 10000
NC = 2
NS = 16
L = 16

DH = D // NC
SEG_PAD = 10240
RPT = SEG_PAD // NS

IDX_COLS = 128
IDX_ROWS = NE // IDX_COLS
BASE_ROWS = IDX_ROWS // NS
EXTRA = IDX_ROWS - BASE_ROWS * NS
RPI = 3
STEPS = BASE_ROWS // RPI
CHUNK = RPI * IDX_COLS


def _sc_body(msg_hbm, idx_hbm, out0_hbm, out1_hbm, acc, cstage,
             buf0, buf1, ib0, ib1, counts, cpart, recip, zbuf,
             si0, sm0, si1, sm1, sa0, sa1):
    c = lax.axis_index("c")
    s = lax.axis_index("s")
    col0 = c * DH
    seg0 = s * RPT
    row_base = s * BASE_ROWS

    zero16 = jnp.zeros((L,), jnp.float32)
    ones16 = jnp.full((L,), 1.0, jnp.float32)

    for i in range(L):
        for j in range(DH // L):
            zbuf[i, pl.ds(j * L, L)] = zero16
    def _zc(i, carry):
        counts[pl.ds(i * L, L)] = zero16
        return carry
    lax.fori_loop(0, SEG_PAD // L, _zc, None)
    def _za(q, carry):
        pltpu.sync_copy(zbuf, acc.at[pl.ds(seg0 + q * L, L)])
        return carry
    lax.fori_loop(0, RPT // L, _za, None)
    plsc.subcore_barrier()

    def _start(row, ib, buf, si, sm):
        pltpu.async_copy(idx_hbm.at[pl.ds(row, RPI)], ib, si)
        pltpu.async_copy(
            msg_hbm.at[pl.ds(row * IDX_COLS, CHUNK), pl.ds(col0, DH)],
            buf, sm)

    def _wait(row, ib, buf, si, sm):
        pltpu.make_async_copy(idx_hbm.at[pl.ds(row, RPI)], ib, si).wait()
        pltpu.make_async_copy(
            msg_hbm.at[pl.ds(row * IDX_COLS, CHUNK), pl.ds(col0, DH)],
            buf, sm).wait()

    def _adds(ib, buf, sa):
        descs = []
        for j in range(RPI):
            descs.append(pltpu.async_copy(
                buf.at[pl.ds(j * IDX_COLS, IDX_COLS)],
                acc.at[ib.at[j]], sa, add=True))
        return descs

    def _count(ib):
        for j in range(RPI):
            for q in range(IDX_COLS // L):
                iv = ib[j, pl.ds(q * L, L)]
                plsc.addupdate_scatter(counts, [iv], ones16)

    _start(row_base, ib0, buf0, si0, sm0)
    def _pair(p, carry):
        row_a = row_base + (2 * p) * RPI
        _start(row_a + RPI, ib1, buf1, si1, sm1)
        _wait(row_a, ib0, buf0, si0, sm0)
        d0 = _adds(ib0, buf0, sa0)
        _count(ib0)
        for d in d0:
            d.wait()

        @pl.when(p < STEPS // 2 - 1)
        def _():
            _start(row_a + 2 * RPI, ib0, buf0, si0, sm0)
        _wait(row_a + RPI, ib1, buf1, si1, sm1)
        d1 = _adds(ib1, buf1, sa1)
        _count(ib1)
        for d in d1:
            d.wait()
        return carry
    lax.fori_loop(0, STEPS // 2, _pair, None)

    @pl.when(s < EXTRA)
    def _extra():
        row = NS * BASE_ROWS + s
        e0 = row * IDX_COLS
        pltpu.sync_copy(idx_hbm.at[pl.ds(row, 1)], ib0.at[pl.ds(0, 1)])
        pltpu.sync_copy(msg_hbm.at[pl.ds(e0, IDX_COLS), pl.ds(col0, DH)],
                        buf0.at[pl.ds(0, IDX_COLS)])
        pltpu.sync_copy(buf0.at[pl.ds(0, IDX_COLS)], acc.at[ib0.at[0]],
                        add=True)
        for q in range(IDX_COLS // L):
            iv = ib0[0, pl.ds(q * L, L)]
            plsc.addupdate_scatter(counts, [iv], ones16)

    pltpu.sync_copy(counts, cstage.at[s])
    plsc.subcore_barrier()

    pltpu.sync_copy(cstage.at[:, pl.ds(seg0, RPT)], cpart)
    def _merge(r, carry):
        tot = zero16
        for t_ in range(NS):
            tot = tot + cpart[t_, pl.ds(r * L, L)]
        recip[pl.ds(r * L, L)] = ones16 / jnp.maximum(tot, ones16)
        return carry
    lax.fori_loop(0, RPT // L, _merge, None)

    for start, n in ((0, CHUNK), (CHUNK, RPT - CHUNK)):
        pltpu.sync_copy(acc.at[pl.ds(seg0 + start, n)], buf0.at[pl.ds(0, n)])
        def _div(r, carry, start=start):
            rv = plsc.load_gather(recip,
                                  [jnp.full((L,), start + r, jnp.int32)])
            for j in range(DH // L):
                buf0[r, pl.ds(j * L, L)] = buf0[r, pl.ds(j * L, L)] * rv
            return carry
        lax.fori_loop(0, n, _div, None)

        @pl.when(c == 0)
        def _w0(start=start, n=n):
            pltpu.sync_copy(buf0.at[pl.ds(0, n)],
                            out0_hbm.at[pl.ds(seg0 + start, n)])

        @pl.when(c == 1)
        def _w1(start=start, n=n):
            pltpu.sync_copy(buf0.at[pl.ds(0, n)],
                            out1_hbm.at[pl.ds(seg0 + start, n)])


@jax.jit
def kernel(msg, index, t):
    del t
    idx2d = index.astype(jnp.int32).reshape(IDX_ROWS, IDX_COLS)
    mesh = plsc.VectorSubcoreMesh(core_axis_name="c", subcore_axis_name="s",
                                  num_cores=NC, num_subcores=NS)
    out = pl.kernel(
        _sc_body,
        out_type=(jax.ShapeDtypeStruct((SEG_PAD, DH), jnp.float32),
                  jax.ShapeDtypeStruct((SEG_PAD, DH), jnp.float32)),
        mesh=mesh,
        compiler_params=pltpu.CompilerParams(use_tc_tiling_on_sc=False,
                                             needs_layout_passes=False),
        scratch_types=[
            pltpu.VMEM_SHARED((SEG_PAD, DH), jnp.float32),
            pltpu.VMEM_SHARED((NS, SEG_PAD), jnp.float32),
            pltpu.VMEM((CHUNK, DH), jnp.float32),
            pltpu.VMEM((CHUNK, DH), jnp.float32),
            pltpu.VMEM((RPI, IDX_COLS), jnp.int32),
            pltpu.VMEM((RPI, IDX_COLS), jnp.int32),
            pltpu.VMEM((SEG_PAD,), jnp.float32),
            pltpu.VMEM((NS, RPT), jnp.float32),
            pltpu.VMEM((RPT,), jnp.float32),
            pltpu.VMEM((L, DH), jnp.float32),
            pltpu.SemaphoreType.DMA,
            pltpu.SemaphoreType.DMA,
            pltpu.SemaphoreType.DMA,
            pltpu.SemaphoreType.DMA,
            pltpu.SemaphoreType.DMA,
            pltpu.SemaphoreType.DMA,
        ],
    )(msg, idx2d)
    return jnp.concatenate([out[0][:NSEG], out[1][:NSEG]], axis=1)


# speedup vs baseline: 10.3177x; 1.0262x over previous
import jax
import jax.numpy as jnp
from jax import lax
from jax.experimental import pallas as pl
from jax.experimental.pallas import tpu as pltpu
from jax.experimental.pallas import tpu_sc as plsc

NE = 320000
D = 128
NSEG = 10000
NC = 2
NS = 16
L = 16

DH = D // NC
SEG_PAD = 10240
RPT = SEG_PAD // NS

IDX_COLS = 128
IDX_ROWS = NE // IDX_COLS
BASE_ROWS = IDX_ROWS // NS
EXTRA = IDX_ROWS - BASE_ROWS * NS
RPI = 2
STEPS = BASE_ROWS // RPI
SUPER = STEPS // 3
CHUNK = RPI * IDX_COLS


def _sc_body(msg_hbm, idx_hbm, out0_hbm, out1_hbm, acc, cstage,
             b0, b1, b2, i0, i1, i2, counts, cpart, recip, zbuf,
             si0, sm0, si1, sm1, si2, sm2, sa0, sa1, sa2):
    bufs = (b0, b1, b2)
    ibs = (i0, i1, i2)
    sis = (si0, si1, si2)
    sms = (sm0, sm1, sm2)
    sas = (sa0, sa1, sa2)

    c = lax.axis_index("c")
    s = lax.axis_index("s")
    col0 = c * DH
    seg0 = s * RPT
    row_base = s * BASE_ROWS

    zero16 = jnp.zeros((L,), jnp.float32)
    ones16 = jnp.full((L,), 1.0, jnp.float32)

    for i in range(L):
        for j in range(DH // L):
            zbuf[i, pl.ds(j * L, L)] = zero16
    def _zc(i, carry):
        counts[pl.ds(i * L, L)] = zero16
        return carry
    lax.fori_loop(0, SEG_PAD // L, _zc, None)
    def _za(q, carry):
        pltpu.sync_copy(zbuf, acc.at[pl.ds(seg0 + q * L, L)])
        return carry
    lax.fori_loop(0, RPT // L, _za, None)
    plsc.subcore_barrier()

    def _load(row, bi):
        pltpu.async_copy(idx_hbm.at[pl.ds(row, RPI)], ibs[bi], sis[bi])
        pltpu.async_copy(
            msg_hbm.at[pl.ds(row * IDX_COLS, CHUNK), pl.ds(col0, DH)],
            bufs[bi], sms[bi])

    def _wait_load(row, bi):
        pltpu.make_async_copy(idx_hbm.at[pl.ds(row, RPI)], ibs[bi],
                              sis[bi]).wait()
        pltpu.make_async_copy(
            msg_hbm.at[pl.ds(row * IDX_COLS, CHUNK), pl.ds(col0, DH)],
            bufs[bi], sms[bi]).wait()

    def _fire(bi):
        for j in range(RPI):
            pltpu.async_copy(bufs[bi].at[pl.ds(j * IDX_COLS, IDX_COLS)],
                             acc.at[ibs[bi].at[j]], sas[bi], add=True)

    def _drain(bi):
        for j in range(RPI):
            pltpu.make_async_copy(bufs[bi].at[pl.ds(j * IDX_COLS, IDX_COLS)],
                                  acc.at[ibs[bi].at[j]], sas[bi]).wait()

    def _count(bi):
        for j in range(RPI):
            for q in range(IDX_COLS // L):
                iv = ibs[bi][j, pl.ds(q * L, L)]
                plsc.addupdate_scatter(counts, [iv], ones16)

    _load(row_base, 0)
    _load(row_base + RPI, 1)

    _wait_load(row_base, 0)
    _fire(0)
    _count(0)
    _load(row_base + 2 * RPI, 2)

    _wait_load(row_base + RPI, 1)
    _fire(1)
    _count(1)
    _drain(0)
    _load(row_base + 3 * RPI, 0)

    _wait_load(row_base + 2 * RPI, 2)
    _fire(2)
    _count(2)
    _drain(1)
    _load(row_base + 4 * RPI, 1)

    def _super(p, carry):
        for i in range(3):
            step = 3 * p + i
            row = row_base + step * RPI
            _wait_load(row, i)
            _fire(i)
            _count(i)
            nbi = (i + 2) % 3
            if i == 0:
                _drain(nbi)
                _load(row + 2 * RPI, nbi)
            else:
                @pl.when(p < SUPER - 1)
                def _(nbi=nbi, row=row):
                    _drain(nbi)
                    _load(row + 2 * RPI, nbi)
        return carry
    lax.fori_loop(1, SUPER, _super, None)
    _drain(0)
    _drain(1)
    _drain(2)

    @pl.when(s < EXTRA)
    def _extra():
        row = NS * BASE_ROWS + s
        e0 = row * IDX_COLS
        pltpu.sync_copy(idx_hbm.at[pl.ds(row, 1)], i0.at[pl.ds(0, 1)])
        pltpu.sync_copy(msg_hbm.at[pl.ds(e0, IDX_COLS), pl.ds(col0, DH)],
                        b0.at[pl.ds(0, IDX_COLS)])
        pltpu.sync_copy(b0.at[pl.ds(0, IDX_COLS)], acc.at[i0.at[0]],
                        add=True)
        for q in range(IDX_COLS // L):
            iv = i0[0, pl.ds(q * L, L)]
            plsc.addupdate_scatter(counts, [iv], ones16)

    pltpu.sync_copy(counts, cstage.at[s])
    plsc.subcore_barrier()

    pltpu.sync_copy(cstage.at[:, pl.ds(seg0, RPT)], cpart)
    def _merge(r, carry):
        tot = zero16
        for t_ in range(NS):
            tot = tot + cpart[t_, pl.ds(r * L, L)]
        recip[pl.ds(r * L, L)] = ones16 / jnp.maximum(tot, ones16)
        return carry
    lax.fori_loop(0, RPT // L, _merge, None)

    for start, n in ((0, CHUNK), (CHUNK, CHUNK), (2 * CHUNK, RPT - 2 * CHUNK)):
        pltpu.sync_copy(acc.at[pl.ds(seg0 + start, n)], b0.at[pl.ds(0, n)])
        def _div(r, carry, start=start):
            rv = plsc.load_gather(recip,
                                  [jnp.full((L,), start + r, jnp.int32)])
            for j in range(DH // L):
                b0[r, pl.ds(j * L, L)] = b0[r, pl.ds(j * L, L)] * rv
            return carry
        lax.fori_loop(0, n, _div, None)

        @pl.when(c == 0)
        def _w0(start=start, n=n):
            pltpu.sync_copy(b0.at[pl.ds(0, n)],
                            out0_hbm.at[pl.ds(seg0 + start, n)])

        @pl.when(c == 1)
        def _w1(start=start, n=n):
            pltpu.sync_copy(b0.at[pl.ds(0, n)],
                            out1_hbm.at[pl.ds(seg0 + start, n)])


@jax.jit
def kernel(msg, index, t):
    del t
    idx2d = index.astype(jnp.int32).reshape(IDX_ROWS, IDX_COLS)
    mesh = plsc.VectorSubcoreMesh(core_axis_name="c", subcore_axis_name="s",
                                  num_cores=NC, num_subcores=NS)
    out = pl.kernel(
        _sc_body,
        out_type=(jax.ShapeDtypeStruct((SEG_PAD, DH), jnp.float32),
                  jax.ShapeDtypeStruct((SEG_PAD, DH), jnp.float32)),
        mesh=mesh,
        compiler_params=pltpu.CompilerParams(use_tc_tiling_on_sc=False,
                                             needs_layout_passes=False),
        scratch_types=[
            pltpu.VMEM_SHARED((SEG_PAD, DH), jnp.float32),
            pltpu.VMEM_SHARED((NS, SEG_PAD), jnp.float32),
            pltpu.VMEM((CHUNK, DH), jnp.float32),
            pltpu.VMEM((CHUNK, DH), jnp.float32),
            pltpu.VMEM((CHUNK, DH), jnp.float32),
            pltpu.VMEM((RPI, IDX_COLS), jnp.int32),
            pltpu.VMEM((RPI, IDX_COLS), jnp.int32),
            pltpu.VMEM((RPI, IDX_COLS), jnp.int32),
            pltpu.VMEM((SEG_PAD,), jnp.float32),
            pltpu.VMEM((NS, RPT), jnp.float32),
            pltpu.VMEM((RPT,), jnp.float32),
            pltpu.VMEM((L, DH), jnp.float32),
            pltpu.SemaphoreType.DMA,
            pltpu.SemaphoreType.DMA,
            pltpu.SemaphoreType.DMA,
            pltpu.SemaphoreType.DMA,
            pltpu.SemaphoreType.DMA,
            pltpu.SemaphoreType.DMA,
            pltpu.SemaphoreType.DMA,
            pltpu.SemaphoreType.DMA,
            pltpu.SemaphoreType.DMA,
        ],
    )(msg, idx2d)
    return jnp.concatenate([out[0][:NSEG], out[1][:NSEG]], axis=1)
